# baseline (speedup 1.0000x reference)
import functools

import jax
import jax.numpy as jnp
from jax import lax
from jax.experimental import pallas as pl
from jax.experimental.pallas import tpu as pltpu
from jax.experimental.pallas import tpu_sc as plsc

EMBED = 768
HID = 512
K = 5
NLAB = 8192
SPEC = 256
N = 4096
BN = 512

NEG = float("-inf")


def _tc_body(x_ref, lab_ref, sw1_ref, sb1_ref, sw2_ref, sb2_ref,
             aw1x_ref, aw1s_ref, ab1_ref, aw2_ref, ab2_ref,
             cs_ref, ids_ref, ab_ref, dom_ref, wexp_ref):
    x = x_ref[...]
    s = jnp.dot(x, lab_ref[...],
                preferred_element_type=jnp.float32)
    col = lax.broadcasted_iota(jnp.int32, (BN, NLAB), 1)

    lanef = lax.broadcasted_iota(jnp.int32, (BN, 128), 1).astype(jnp.float32)
    vals, idxs = [], []
    for k in range(K):
        best = jnp.full((BN, 128), NEG, jnp.float32)
        btile = jnp.zeros((BN, 128), jnp.float32)
        for t in range(NLAB // 128):
            v = s[:, t * 128:(t + 1) * 128]
            gt = v > best
            best = jnp.where(gt, v, best)
            btile = jnp.where(gt, jnp.float32(t), btile)
        m = jnp.max(best, axis=1, keepdims=True)
        colc = btile * 128.0 + lanef
        idxf = jnp.min(jnp.where(best == m, colc, jnp.float32(NLAB)),
                       axis=1, keepdims=True)
        idx = idxf.astype(jnp.int32)
        vals.append(m)
        idxs.append(idx)
        if k < K - 1:
            s = jnp.where(col == idx, NEG, s)
    cand_sims = jnp.concatenate(vals, axis=1)
    cand_ids = jnp.concatenate(idxs, axis=1)

    h1 = jnp.maximum(
        jnp.dot(x, sw1_ref[...], preferred_element_type=jnp.float32)
        + sb1_ref[...], 0.0)
    scores = (jnp.dot(h1, sw2_ref[...], preferred_element_type=jnp.float32)
              + sb2_ref[...])

    h2 = jnp.maximum(
        jnp.dot(x, aw1x_ref[...], preferred_element_type=jnp.float32)
        + jnp.dot(scores, aw1s_ref[...], preferred_element_type=jnp.float32)
        + ab1_ref[...], 0.0)
    logits = (jnp.dot(h2, aw2_ref[...], preferred_element_type=jnp.float32)
              + ab2_ref[...])
    mx = jnp.max(logits, axis=1, keepdims=True)
    e = jnp.exp(logits - mx)
    ab = e / jnp.sum(e, axis=1, keepdims=True)

    ab5 = ab[:, :K]
    lane = lax.broadcasted_iota(jnp.int32, (BN, K), 1)
    am = jnp.max(ab5, axis=1, keepdims=True)
    slot = jnp.min(jnp.where(ab5 == am, lane, K), axis=1, keepdims=True)
    dom = jnp.max(jnp.where(lane == slot, cand_ids, -1), axis=1, keepdims=True)

    cs_ref[...] = cand_sims
    ids_ref[...] = cand_ids
    ab_ref[...] = ab
    dom_ref[...] = dom
    wexp_ref[...] = jnp.broadcast_to(ab5[:, :, None], (BN, K, 16)).reshape(BN, K * 16)


def _transpose_body(x_ref, o_ref):
    o_ref[...] = x_ref[...].T


def _transpose_lab(lab):
    tile = 1024
    return pl.pallas_call(
        _transpose_body,
        grid=(NLAB // tile,),
        in_specs=[pl.BlockSpec((tile, EMBED), lambda r: (r, 0))],
        out_specs=pl.BlockSpec((EMBED, tile), lambda r: (0, r)),
        out_shape=jax.ShapeDtypeStruct((EMBED, NLAB), jnp.float32),
    )(lab)


def _tc_head(flat, lab, sw1, sb1, sw2, sb2, aw1x, aw1s, ab1, aw2, ab2):
    nblocks = N // BN
    full = lambda r: (0, 0)
    prev = lambda r: (r, 0)
    grid = (nblocks,)
    in_specs = [
            pl.BlockSpec((BN, EMBED), lambda r: (r, 0)),
            pl.BlockSpec((EMBED, NLAB), full),
            pl.BlockSpec((EMBED, HID), full),
            pl.BlockSpec((1, HID), full),
            pl.BlockSpec((HID, K), full),
            pl.BlockSpec((1, K), full),
            pl.BlockSpec((EMBED, HID), full),
            pl.BlockSpec((K, HID), full),
            pl.BlockSpec((1, HID), full),
            pl.BlockSpec((HID, K + 2), full),
            pl.BlockSpec((1, K + 2), full),
    ]
    out_specs = [
            pl.BlockSpec((BN, K), prev),
            pl.BlockSpec((BN, K), prev),
            pl.BlockSpec((BN, K + 2), prev),
            pl.BlockSpec((BN, 1), prev),
            pl.BlockSpec((BN, K * 16), prev),
    ]
    return pl.pallas_call(
        _tc_body,
        grid=grid,
        in_specs=in_specs,
        out_specs=out_specs,
        out_shape=[
            jax.ShapeDtypeStruct((N, K), jnp.float32),
            jax.ShapeDtypeStruct((N, K), jnp.int32),
            jax.ShapeDtypeStruct((N, K + 2), jnp.float32),
            jax.ShapeDtypeStruct((N, 1), jnp.int32),
            jax.ShapeDtypeStruct((N, K * 16), jnp.float32),
        ],
    )(flat, lab, sw1, sb1, sw2, sb2, aw1x, aw1s, ab1, aw2, ab2)


_NW = 32
_PPW = N // _NW
_CH = 16


def _sc_recon_body(ids_hbm, wexp_hbm, protos_hbm, out_hbm,
                   ids_v, w_v, rows_v, out_v, sem0, sem1):
    c = lax.axis_index("c")
    s = lax.axis_index("s")
    wid = s * 2 + c
    base = wid * _PPW
    sems = (sem0, sem1)
    nch = _PPW // _CH

    pltpu.sync_copy(ids_hbm.at[pl.ds(base * K, _PPW * K)], ids_v)

    def start(ch):
        bb = ch % 2
        pltpu.sync_copy(
            wexp_hbm.at[pl.ds(base * K + ch * _CH * K, _CH * K)], w_v.at[bb])
        return pltpu.async_copy(
            protos_hbm.at[ids_v.at[pl.ds(ch * _CH * K, _CH * K)]],
            rows_v.at[bb], sems[bb])

    cp = start(0)
    for ch in range(nch):
        bb = ch % 2
        nxt = start(ch + 1) if ch + 1 < nch else None
        cp.wait()

        def body(p, carry):
            fp = p * K
            ws = [w_v[bb, fp + k, :] for k in range(K)]
            for c16 in range(SPEC // 16):
                sl = pl.ds(c16 * 16, 16)
                acc = ws[0] * rows_v[bb, p * K, sl]
                for k in range(1, K):
                    acc = acc + ws[k] * rows_v[bb, p * K + k, sl]
                out_v[bb, p, sl] = acc
            return carry

        lax.fori_loop(0, _CH, body, 0)
        pltpu.sync_copy(out_v.at[bb], out_hbm.at[pl.ds(base + ch * _CH, _CH)])
        cp = nxt


def _sc_recon(ids_flat, wexp, protos):
    mesh = plsc.VectorSubcoreMesh(core_axis_name="c", subcore_axis_name="s")
    f = functools.partial(
        pl.kernel,
        mesh=mesh,
        out_type=jax.ShapeDtypeStruct((N, SPEC), jnp.float32),
        scratch_types=[
            pltpu.VMEM((_PPW * K,), jnp.int32),
            pltpu.VMEM((2, _CH * K, 16), jnp.float32),
            pltpu.VMEM((2, _CH * K, SPEC), jnp.float32),
            pltpu.VMEM((2, _CH, SPEC), jnp.float32),
            pltpu.SemaphoreType.DMA,
            pltpu.SemaphoreType.DMA,
        ],
    )(_sc_recon_body)
    return f(ids_flat, wexp, protos)


def kernel(features, lab_embeddings, prototype_spectra,
           score_w1, score_b1, score_w2, score_b2,
           ab_w1, ab_b1, ab_w2, ab_b2):
    b, h, w, cdim = features.shape
    flat = features.reshape(-1, cdim)
    aw1x = ab_w1[:EMBED]
    aw1s = ab_w1[EMBED:]

    cand_sims, cand_ids, abundances, dominant, wexp = _tc_head(
        flat, _transpose_lab(lab_embeddings),
        score_w1, score_b1.reshape(1, -1), score_w2, score_b2.reshape(1, -1),
        aw1x, aw1s, ab_b1.reshape(1, -1), ab_w2, ab_b2.reshape(1, -1))

    ids_flat = cand_ids.reshape(-1)
    recon = _sc_recon(ids_flat, wexp.reshape(N * K, 16), prototype_spectra)

    return (dominant.reshape(b, h, w),
            abundances.reshape(b, h, w, -1),
            recon.reshape(b, h, w, -1),
            cand_sims.reshape(b, h, w, -1))


# speedup vs baseline: 1.0084x; 1.0084x over previous
import functools

import jax
import jax.numpy as jnp
from jax import lax
from jax.experimental import pallas as pl
from jax.experimental.pallas import tpu as pltpu
from jax.experimental.pallas import tpu_sc as plsc

EMBED = 768
HID = 512
K = 5
NLAB = 8192
SPEC = 256
N = 4096
BN = 512

NEG = float("-inf")


def _tc_body(x_ref, lab_ref, sw1_ref, sb1_ref, sw2_ref, sb2_ref,
             aw1x_ref, aw1s_ref, ab1_ref, aw2_ref, ab2_ref,
             cs_ref, ids_ref, ab_ref, dom_ref, wexp_ref):
    x = x_ref[...]
    s = jnp.dot(x, lab_ref[...],
                preferred_element_type=jnp.float32)
    col = lax.broadcasted_iota(jnp.int32, (BN, NLAB), 1)

    lanef = lax.broadcasted_iota(jnp.int32, (BN, 128), 1).astype(jnp.float32)
    vals, idxs = [], []
    for k in range(K):
        best = jnp.full((BN, 128), NEG, jnp.float32)
        btile = jnp.zeros((BN, 128), jnp.float32)
        for t in range(NLAB // 128):
            v = s[:, t * 128:(t + 1) * 128]
            gt = v > best
            best = jnp.where(gt, v, best)
            btile = jnp.where(gt, jnp.float32(t), btile)
        m = jnp.max(best, axis=1, keepdims=True)
        colc = btile * 128.0 + lanef
        idxf = jnp.min(jnp.where(best == m, colc, jnp.float32(NLAB)),
                       axis=1, keepdims=True)
        idx = idxf.astype(jnp.int32)
        vals.append(m)
        idxs.append(idx)
        if k < K - 1:
            s = jnp.where(col == idx, NEG, s)
    cand_sims = jnp.concatenate(vals, axis=1)
    cand_ids = jnp.concatenate(idxs, axis=1)

    h1 = jnp.maximum(
        jnp.dot(x, sw1_ref[...], preferred_element_type=jnp.float32)
        + sb1_ref[...], 0.0)
    scores = (jnp.dot(h1, sw2_ref[...], preferred_element_type=jnp.float32)
              + sb2_ref[...])

    h2 = jnp.maximum(
        jnp.dot(x, aw1x_ref[...], preferred_element_type=jnp.float32)
        + jnp.dot(scores, aw1s_ref[...], preferred_element_type=jnp.float32)
        + ab1_ref[...], 0.0)
    logits = (jnp.dot(h2, aw2_ref[...], preferred_element_type=jnp.float32)
              + ab2_ref[...])
    mx = jnp.max(logits, axis=1, keepdims=True)
    e = jnp.exp(logits - mx)
    ab = e / jnp.sum(e, axis=1, keepdims=True)

    ab5 = ab[:, :K]
    lane = lax.broadcasted_iota(jnp.int32, (BN, K), 1)
    am = jnp.max(ab5, axis=1, keepdims=True)
    slot = jnp.min(jnp.where(ab5 == am, lane, K), axis=1, keepdims=True)
    dom = jnp.max(jnp.where(lane == slot, cand_ids, -1), axis=1, keepdims=True)

    cs_ref[...] = cand_sims
    ids_ref[...] = cand_ids
    ab_ref[...] = ab
    dom_ref[...] = dom
    wexp_ref[...] = jnp.broadcast_to(ab5[:, :, None], (BN, K, 16)).reshape(BN, K * 16)


def _tc_head(flat, lab, sw1, sb1, sw2, sb2, aw1x, aw1s, ab1, aw2, ab2):
    nblocks = N // BN
    full = lambda r: (0, 0)
    prev = lambda r: (r, 0)
    grid = (nblocks,)
    in_specs = [
            pl.BlockSpec((BN, EMBED), lambda r: (r, 0)),
            pl.BlockSpec((EMBED, NLAB), full),
            pl.BlockSpec((EMBED, HID), full),
            pl.BlockSpec((1, HID), full),
            pl.BlockSpec((HID, K), full),
            pl.BlockSpec((1, K), full),
            pl.BlockSpec((EMBED, HID), full),
            pl.BlockSpec((K, HID), full),
            pl.BlockSpec((1, HID), full),
            pl.BlockSpec((HID, K + 2), full),
            pl.BlockSpec((1, K + 2), full),
    ]
    out_specs = [
            pl.BlockSpec((BN, K), prev),
            pl.BlockSpec((BN, K), prev),
            pl.BlockSpec((BN, K + 2), prev),
            pl.BlockSpec((BN, 1), prev),
            pl.BlockSpec((BN, K * 16), prev),
    ]
    return pl.pallas_call(
        _tc_body,
        grid=grid,
        in_specs=in_specs,
        out_specs=out_specs,
        out_shape=[
            jax.ShapeDtypeStruct((N, K), jnp.float32),
            jax.ShapeDtypeStruct((N, K), jnp.int32),
            jax.ShapeDtypeStruct((N, K + 2), jnp.float32),
            jax.ShapeDtypeStruct((N, 1), jnp.int32),
            jax.ShapeDtypeStruct((N, K * 16), jnp.float32),
        ],
    )(flat, lab, sw1, sb1, sw2, sb2, aw1x, aw1s, ab1, aw2, ab2)


_NW = 32
_PPW = N // _NW
_CH = 16


def _sc_recon_body(ids_hbm, wexp_hbm, protos_hbm, out_hbm,
                   ids_v, w_v, rows_v, out_v, sem0, sem1):
    c = lax.axis_index("c")
    s = lax.axis_index("s")
    wid = s * 2 + c
    base = wid * _PPW
    sems = (sem0, sem1)
    nch = _PPW // _CH

    pltpu.sync_copy(ids_hbm.at[pl.ds(base * K, _PPW * K)], ids_v)

    def start(ch):
        bb = ch % 2
        pltpu.sync_copy(
            wexp_hbm.at[pl.ds(base * K + ch * _CH * K, _CH * K)], w_v.at[bb])
        return pltpu.async_copy(
            protos_hbm.at[ids_v.at[pl.ds(ch * _CH * K, _CH * K)]],
            rows_v.at[bb], sems[bb])

    cp = start(0)
    for ch in range(nch):
        bb = ch % 2
        nxt = start(ch + 1) if ch + 1 < nch else None
        cp.wait()

        def body(p, carry):
            fp = p * K
            ws = [w_v[bb, fp + k, :] for k in range(K)]
            for c16 in range(SPEC // 16):
                sl = pl.ds(c16 * 16, 16)
                acc = ws[0] * rows_v[bb, p * K, sl]
                for k in range(1, K):
                    acc = acc + ws[k] * rows_v[bb, p * K + k, sl]
                out_v[bb, p, sl] = acc
            return carry

        lax.fori_loop(0, _CH, body, 0)
        pltpu.sync_copy(out_v.at[bb], out_hbm.at[pl.ds(base + ch * _CH, _CH)])
        cp = nxt


def _sc_recon(ids_flat, wexp, protos):
    mesh = plsc.VectorSubcoreMesh(core_axis_name="c", subcore_axis_name="s")
    f = functools.partial(
        pl.kernel,
        mesh=mesh,
        out_type=jax.ShapeDtypeStruct((N, SPEC), jnp.float32),
        scratch_types=[
            pltpu.VMEM((_PPW * K,), jnp.int32),
            pltpu.VMEM((2, _CH * K, 16), jnp.float32),
            pltpu.VMEM((2, _CH * K, SPEC), jnp.float32),
            pltpu.VMEM((2, _CH, SPEC), jnp.float32),
            pltpu.SemaphoreType.DMA,
            pltpu.SemaphoreType.DMA,
        ],
    )(_sc_recon_body)
    return f(ids_flat, wexp, protos)


def kernel(features, lab_embeddings, prototype_spectra,
           score_w1, score_b1, score_w2, score_b2,
           ab_w1, ab_b1, ab_w2, ab_b2):
    b, h, w, cdim = features.shape
    flat = features.reshape(-1, cdim)
    aw1x = ab_w1[:EMBED]
    aw1s = ab_w1[EMBED:]

    cand_sims, cand_ids, abundances, dominant, wexp = _tc_head(
        flat, lab_embeddings.T,
        score_w1, score_b1.reshape(1, -1), score_w2, score_b2.reshape(1, -1),
        aw1x, aw1s, ab_b1.reshape(1, -1), ab_w2, ab_b2.reshape(1, -1))

    ids_flat = cand_ids.reshape(-1)
    recon = _sc_recon(ids_flat, wexp.reshape(N * K, 16), prototype_spectra)

    return (dominant.reshape(b, h, w),
            abundances.reshape(b, h, w, -1),
            recon.reshape(b, h, w, -1),
            cand_sims.reshape(b, h, w, -1))
